# Initial kernel scaffold; baseline (speedup 1.0000x reference)
import jax
import jax.numpy as jnp
from jax.experimental import pallas as pl


def kernel(x):
    raise NotImplementedError("write your pallas kernel here")


# speedup vs baseline: 4.4051x; 4.4051x over previous
import functools

import jax
import jax.numpy as jnp
from jax import lax
from jax.experimental import pallas as pl
from jax.experimental.pallas import tpu as pltpu
from jax.experimental.pallas import tpu_sc as plsc

B, S, D = 4, 8192, 1024
K = 2048
ROWS = 1024


def _sumsq_body(x_ref, out_ref):
    x = x_ref[0]
    p = None
    for c in range(8):
        xc = x[:, c * 128:(c + 1) * 128]
        sq = xc * xc
        p = sq if p is None else sq + p
    a = None
    for t in range(16):
        pt = p[:, 8 * t:8 * t + 8]
        a = pt if a is None else pt + a
    b1 = a[:, 0:4] + a[:, 4:8]
    b2 = b1[:, 0:2] + b1[:, 2:4]
    s = b2[:, 0:1] + b2[:, 1:2]
    out_ref[0] = s


def _sumsq(x):
    xr = x.reshape(B * S // ROWS, ROWS, D)
    out = pl.pallas_call(
        _sumsq_body,
        grid=(B * S // ROWS,),
        in_specs=[pl.BlockSpec((1, ROWS, D), lambda i: (i, 0, 0))],
        out_specs=pl.BlockSpec((1, ROWS, 1), lambda i: (i, 0, 0)),
        out_shape=jax.ShapeDtypeStruct((B * S // ROWS, ROWS, 1), jnp.float32),
    )(xr)
    return out.reshape(B, S)


def _topk_body(n_ref, out_ref):
    k = n_ref[...]
    row = lax.broadcasted_iota(jnp.int32, (B, 64, 128), 1)
    col = lax.broadcasted_iota(jnp.int32, (B, 64, 128), 2)
    tok = row * 128 + col
    idx = tok
    for p in range(13):
        asc = ((tok >> (p + 1)) & 1) == 1
        for j in range(p, -1, -1):
            m = 1 << j
            am_low = (tok & m) == 0
            if j < 7:
                ax, sh = 2, m
            else:
                ax, sh = 1, m >> 7
            kp = jnp.where(am_low, jnp.roll(k, -sh, axis=ax),
                           jnp.roll(k, sh, axis=ax))
            ip = jnp.where(am_low, jnp.roll(idx, -sh, axis=ax),
                           jnp.roll(idx, sh, axis=ax))
            pw = (kp > k) | ((kp == k) & (ip < idx))
            take = (pw == am_low) != asc
            k = jnp.where(take, kp, k)
            idx = jnp.where(take, ip, idx)
    b = lax.broadcasted_iota(jnp.int32, (B, 16, 128), 0)
    out_ref[...] = idx[:, 0:16, :] + b * S


def _topk(norms):
    out = pl.pallas_call(
        _topk_body,
        in_specs=[pl.BlockSpec((B, 64, 128), lambda: (0, 0, 0))],
        out_specs=pl.BlockSpec((B, 16, 128), lambda: (0, 0, 0)),
        out_shape=jax.ShapeDtypeStruct((B, 16, 128), jnp.int32),
    )(norms.reshape(B, 64, 128))
    return out.reshape(B * K)


def _sc_gather(xf, gid):
    info = plsc.get_sparse_core_info()
    nw = info.num_cores * info.num_subcores
    rows_w = (B * K) // nw
    chunk = 64
    mesh = plsc.VectorSubcoreMesh(core_axis_name="c", subcore_axis_name="s")

    @functools.partial(
        pl.kernel, mesh=mesh,
        out_type=jax.ShapeDtypeStruct((B * K, D), jnp.float32),
        scratch_types=[
            pltpu.VMEM((chunk,), jnp.int32),
            pltpu.VMEM((chunk, D), jnp.float32),
            pltpu.SemaphoreType.DMA,
        ],
    )
    def gather_k(x_hbm, gid_hbm, out_hbm, idx_v, rows_v, sem):
        wid = lax.axis_index("s") * info.num_cores + lax.axis_index("c")
        for ch in range(rows_w // chunk):
            base = wid * rows_w + ch * chunk
            pltpu.sync_copy(gid_hbm.at[pl.ds(base, chunk)], idx_v)
            pltpu.async_copy(x_hbm.at[idx_v], rows_v, sem).wait()
            pltpu.sync_copy(rows_v, out_hbm.at[pl.ds(base, chunk)])

    return gather_k(xf, gid)


def kernel(x):
    ss = _sumsq(x)
    norms = jnp.sqrt(ss)
    gid = _topk(norms)
    out = _sc_gather(x.reshape(B * S, D), gid)
    return out.reshape(B, K, D)


# speedup vs baseline: 6.6539x; 1.5105x over previous
import functools

import jax
import jax.numpy as jnp
from jax import lax
from jax.experimental import pallas as pl
from jax.experimental.pallas import tpu as pltpu
from jax.experimental.pallas import tpu_sc as plsc

B, S, D = 4, 8192, 1024
K = 2048
ROWS = 1024


def _sumsq_body(x_ref, out_ref):
    x = x_ref[0]
    p = None
    for c in range(8):
        xc = x[:, c * 128:(c + 1) * 128]
        sq = xc * xc
        p = sq if p is None else sq + p
    pt = jnp.transpose(p)
    a = None
    for t in range(16):
        at = pt[8 * t:8 * t + 8, :]
        a = at if a is None else at + a
    b1 = a[0:4, :] + a[4:8, :]
    b2 = b1[0:2, :] + b1[2:4, :]
    s = b2[0:1, :] + b2[1:2, :]
    out_ref[0] = s


def _sumsq(x):
    xr = x.reshape(B * S // ROWS, ROWS, D)
    out = pl.pallas_call(
        _sumsq_body,
        grid=(B * S // ROWS,),
        in_specs=[pl.BlockSpec((1, ROWS, D), lambda i: (i, 0, 0))],
        out_specs=pl.BlockSpec((1, 1, ROWS), lambda i: (i, 0, 0)),
        out_shape=jax.ShapeDtypeStruct((B * S // ROWS, 1, ROWS), jnp.float32),
    )(xr)
    return out.reshape(B, S)


def _topk_body(n_ref, out_ref):
    k = n_ref[...]
    row = lax.broadcasted_iota(jnp.int32, (B, 64, 128), 1)
    col = lax.broadcasted_iota(jnp.int32, (B, 64, 128), 2)
    tok = row * 128 + col
    idx = tok
    for p in range(13):
        asc = ((tok >> (p + 1)) & 1) == 1
        for j in range(p, -1, -1):
            m = 1 << j
            am_low = (tok & m) == 0
            if j < 7:
                ax, sh = 2, m
            else:
                ax, sh = 1, m >> 7
            kp = jnp.where(am_low, jnp.roll(k, -sh, axis=ax),
                           jnp.roll(k, sh, axis=ax))
            ip = jnp.where(am_low, jnp.roll(idx, -sh, axis=ax),
                           jnp.roll(idx, sh, axis=ax))
            pw = (kp > k) | ((kp == k) & (ip < idx))
            take = (pw == am_low) != asc
            k = jnp.where(take, kp, k)
            idx = jnp.where(take, ip, idx)
    b = lax.broadcasted_iota(jnp.int32, (B, 16, 128), 0)
    out_ref[...] = idx[:, 0:16, :] + b * S


def _topk(norms):
    out = pl.pallas_call(
        _topk_body,
        in_specs=[pl.BlockSpec((B, 64, 128), lambda: (0, 0, 0))],
        out_specs=pl.BlockSpec((B, 16, 128), lambda: (0, 0, 0)),
        out_shape=jax.ShapeDtypeStruct((B, 16, 128), jnp.int32),
    )(norms.reshape(B, 64, 128))
    return out.reshape(B * K)


def _sc_gather(xf, gid):
    info = plsc.get_sparse_core_info()
    nw = info.num_cores * info.num_subcores
    rows_w = (B * K) // nw
    chunk = 64
    mesh = plsc.VectorSubcoreMesh(core_axis_name="c", subcore_axis_name="s")

    @functools.partial(
        pl.kernel, mesh=mesh,
        out_type=jax.ShapeDtypeStruct((B * K, D), jnp.float32),
        scratch_types=[
            pltpu.VMEM((chunk,), jnp.int32),
            pltpu.VMEM((chunk, D), jnp.float32),
            pltpu.SemaphoreType.DMA,
        ],
    )
    def gather_k(x_hbm, gid_hbm, out_hbm, idx_v, rows_v, sem):
        wid = lax.axis_index("s") * info.num_cores + lax.axis_index("c")
        for ch in range(rows_w // chunk):
            base = wid * rows_w + ch * chunk
            pltpu.sync_copy(gid_hbm.at[pl.ds(base, chunk)], idx_v)
            pltpu.async_copy(x_hbm.at[idx_v], rows_v, sem).wait()
            pltpu.sync_copy(rows_v, out_hbm.at[pl.ds(base, chunk)])

    return gather_k(xf, gid)


def kernel(x):
    ss = _sumsq(x)
    norms = jnp.sqrt(ss)
    gid = _topk(norms)
    out = _sc_gather(x.reshape(B * S, D), gid)
    return out.reshape(B, K, D)
